# baseline (speedup 1.0000x reference)
import functools

import jax
import jax.numpy as jnp
from jax import lax
from jax.experimental import pallas as pl
from jax.experimental.pallas import tpu as pltpu
from jax.experimental.pallas import tpu_sc as plsc

B, N, M = 32, 512, 512
L = 16
NC, NS = 2, 16
ROWS_BLK = 64
PIPE = 4
PIPE_ROWS = N // PIPE
BIG = 2**30


def _p1_rows(rowbuf, pen, rem, row0, lane_iota):
    n_grp = 4
    per_grp = M // L // n_grp

    def row_body(r, carry):
        i = row0 + r
        accs = []
        for g in range(n_grp):
            bv = jnp.full((L,), -jnp.inf, jnp.float32)
            bi = jnp.zeros((L,), jnp.int32)
            for k in range(per_grp):
                kk = g * per_grp + k
                v = rowbuf[r, pl.ds(kk * L, L)] + pen[pl.ds(kk * L, L)]
                gt = v > bv
                bv = jnp.where(gt, v, bv)
                bi = jnp.where(gt, lane_iota + (kk * L), bi)
            accs.append((bv, bi))
        while len(accs) > 1:
            nxt = []
            for (av, ai), (bv, bi) in zip(accs[::2], accs[1::2]):
                gt = bv > av
                nxt.append((jnp.where(gt, bv, av), jnp.where(gt, bi, ai)))
            accs = nxt
        best_v, best_i = accs[0]
        mx = jnp.max(best_v)
        cand = jnp.where(best_v == mx, best_i, jnp.int32(BIG))
        idx = jnp.min(cand)
        idxv = jnp.full((L,), idx, jnp.int32)
        lane0 = lane_iota == 0
        plsc.store_scatter(
            pen, [idxv], jnp.full((L,), -jnp.inf, jnp.float32), mask=lane0)
        plsc.store_scatter(
            rem, [idxv], jnp.full((L,), i, jnp.int32), mask=lane0)
        return carry

    lax.fori_loop(0, ROWS_BLK, row_body, 0)


def _p1_block_body(blk):
    def body(s_hbm, pen_in, rem_in, pen_out, rem_out,
             buf0, buf1, pen, rem, sem0, sem1):
        b = lax.axis_index("s") * NC + lax.axis_index("c")
        lane_iota = lax.broadcasted_iota(jnp.int32, (L,), 0)

        if blk == 0:
            for k in range(M // L):
                pen[pl.ds(k * L, L)] = jnp.zeros((L,), jnp.float32)
                rem[pl.ds(k * L, L)] = jnp.full((L,), BIG, jnp.int32)
        else:
            pltpu.sync_copy(pen_in.at[b], pen)
            pltpu.sync_copy(rem_in.at[b], rem)

        bufs = (buf0, buf1)
        sems = (sem0, sem1)
        n_sub = PIPE_ROWS // ROWS_BLK
        row_base = blk * PIPE_ROWS
        copies = [None] * n_sub
        copies[0] = pltpu.async_copy(
            s_hbm.at[b, pl.ds(row_base, ROWS_BLK)], bufs[0], sems[0])
        for sub in range(n_sub):
            copies[sub].wait()
            if sub + 1 < n_sub:
                copies[sub + 1] = pltpu.async_copy(
                    s_hbm.at[b, pl.ds(row_base + (sub + 1) * ROWS_BLK,
                                      ROWS_BLK)],
                    bufs[(sub + 1) % 2], sems[(sub + 1) % 2])
            _p1_rows(bufs[sub % 2], pen, rem,
                     row_base + sub * ROWS_BLK, lane_iota)

        pltpu.sync_copy(pen, pen_out.at[b])
        pltpu.sync_copy(rem, rem_out.at[b])
    return body


def _p1_block(blk, s, pen_state, rem_state):
    mesh = plsc.VectorSubcoreMesh(core_axis_name="c", subcore_axis_name="s")
    kern = functools.partial(
        pl.kernel,
        mesh=mesh,
        out_type=(
            jax.ShapeDtypeStruct((B, M), jnp.float32),
            jax.ShapeDtypeStruct((B, M), jnp.int32),
        ),
        scratch_types=[
            pltpu.VMEM((ROWS_BLK, M), jnp.float32),
            pltpu.VMEM((ROWS_BLK, M), jnp.float32),
            pltpu.VMEM((M,), jnp.float32),
            pltpu.VMEM((M,), jnp.int32),
            pltpu.SemaphoreType.DMA,
            pltpu.SemaphoreType.DMA,
        ],
        compiler_params=pltpu.CompilerParams(needs_layout_passes=False),
        name=f"p1_blk{blk}",
    )(_p1_block_body(blk))
    return kern(s, pen_state, rem_state)


def _p2_kernel_body(blk, *refs):
    if blk == 0:
        s_ref, rem_ref, o_ref = refs
    else:
        _, s_ref, rem_ref, o_ref = refs
    rows = s_ref[0]
    ra = rem_ref[0]
    row_ids = (blk * PIPE_ROWS
               + lax.broadcasted_iota(jnp.int32, (PIPE_ROWS, 1), 0))
    mask = ra >= row_ids
    neg = jnp.where(mask, rows, -jnp.inf)
    mx = jnp.max(neg, axis=1, keepdims=True)
    e = jnp.where(mask, jnp.exp(rows - mx), 0.0)
    o_ref[0] = e / jnp.sum(e, axis=1, keepdims=True)


def _p2_block(blk, out_prev, s, removed):
    rem3 = removed.reshape(B, 1, M)
    blk_spec = pl.BlockSpec((1, PIPE_ROWS, M), lambda bb: (bb, blk, 0))
    in_specs = [
        blk_spec,
        pl.BlockSpec((1, 1, M), lambda bb: (bb, 0, 0)),
    ]
    operands = (s, rem3)
    aliases = {}
    if blk > 0:
        in_specs = [pl.BlockSpec(memory_space=pl.ANY)] + in_specs
        operands = (out_prev,) + operands
        aliases = {0: 0}
    return pl.pallas_call(
        functools.partial(_p2_kernel_body, blk),
        grid=(B,),
        in_specs=in_specs,
        out_specs=blk_spec,
        out_shape=jax.ShapeDtypeStruct((B, N, M), jnp.float32),
        input_output_aliases=aliases,
        name=f"p2_blk{blk}",
    )(*operands)


def kernel(s):
    pen_state = rem_state = None
    out = None
    for blk in range(PIPE):
        if blk == 0:
            zf = jnp.zeros((B, M), jnp.float32)
            zi = jnp.zeros((B, M), jnp.int32)
            pen_state, rem_state = _p1_block(blk, s, zf, zi)
        else:
            pen_state, rem_state = _p1_block(blk, s, pen_state, rem_state)
        out = _p2_block(blk, out, s, rem_state)
    return out


# speedup vs baseline: 1.6243x; 1.6243x over previous
import functools

import jax
import jax.numpy as jnp
from jax import lax
from jax.experimental import pallas as pl
from jax.experimental.pallas import tpu as pltpu
from jax.experimental.pallas import tpu_sc as plsc

B, N, M = 32, 512, 512
L = 16
NC, NS = 2, 16
ROWS_BLK = 64
TC_BATCH = 1
BIG = 2**30


def _p1_body(s_hbm, removed_hbm, buf0, buf1, pen, rem, sem0, sem1):
    b = lax.axis_index("s") * NC + lax.axis_index("c")
    lane_iota = lax.broadcasted_iota(jnp.int32, (L,), 0)

    for k in range(M // L):
        pen[pl.ds(k * L, L)] = jnp.zeros((L,), jnp.float32)
        rem[pl.ds(k * L, L)] = jnp.zeros((L,), jnp.int32)

    bufs = (buf0, buf1)
    sems = (sem0, sem1)
    n_blk = N // ROWS_BLK
    copies = [None] * n_blk
    copies[0] = pltpu.async_copy(
        s_hbm.at[b, pl.ds(0, ROWS_BLK)], bufs[0], sems[0])

    n_grp = 4
    per_grp = M // L // n_grp

    for blk in range(n_blk):
        rowbuf = bufs[blk % 2]
        copies[blk].wait()
        if blk + 1 < n_blk:
            copies[blk + 1] = pltpu.async_copy(
                s_hbm.at[b, pl.ds((blk + 1) * ROWS_BLK, ROWS_BLK)],
                bufs[(blk + 1) % 2], sems[(blk + 1) % 2])

        def row_body(r, carry, rowbuf=rowbuf, blk=blk):
            i = blk * ROWS_BLK + r
            accs = []
            for g in range(n_grp):
                bv = jnp.full((L,), -jnp.inf, jnp.float32)
                bi = jnp.zeros((L,), jnp.int32)
                for k in range(per_grp):
                    kk = g * per_grp + k
                    v = rowbuf[r, pl.ds(kk * L, L)] + pen[pl.ds(kk * L, L)]
                    gt = v > bv
                    bv = jnp.where(gt, v, bv)
                    bi = jnp.where(gt, lane_iota + (kk * L), bi)
                accs.append((bv, bi))
            while len(accs) > 1:
                nxt = []
                for (av, ai), (bv, bi) in zip(accs[::2], accs[1::2]):
                    gt = bv > av
                    nxt.append((jnp.where(gt, bv, av), jnp.where(gt, bi, ai)))
                accs = nxt
            best_v, best_i = accs[0]
            mx = jnp.max(best_v)
            cand = jnp.where(best_v == mx, best_i, jnp.int32(BIG))
            idx = jnp.min(cand)
            idxv = jnp.full((L,), idx, jnp.int32)
            lane0 = lane_iota == 0
            plsc.store_scatter(
                pen, [idxv], jnp.full((L,), -jnp.inf, jnp.float32), mask=lane0)
            plsc.store_scatter(
                rem, [idxv], jnp.full((L,), i, jnp.int32), mask=lane0)
            return carry

        lax.fori_loop(0, ROWS_BLK, row_body, 0)

    pltpu.sync_copy(rem, removed_hbm.at[b])


def _phase1(s):
    mesh = plsc.VectorSubcoreMesh(core_axis_name="c", subcore_axis_name="s")
    kern = functools.partial(
        pl.kernel,
        mesh=mesh,
        out_type=jax.ShapeDtypeStruct((B, M), jnp.int32),
        scratch_types=[
            pltpu.VMEM((ROWS_BLK, M), jnp.float32),
            pltpu.VMEM((ROWS_BLK, M), jnp.float32),
            pltpu.VMEM((M,), jnp.float32),
            pltpu.VMEM((M,), jnp.int32),
            pltpu.SemaphoreType.DMA,
            pltpu.SemaphoreType.DMA,
        ],
        compiler_params=pltpu.CompilerParams(needs_layout_passes=False),
        name="p1_greedy",
    )(_p1_body)
    return kern(s)


def _p2_kernel(s_ref, rem_ref, o_ref):
    rows = s_ref[...]
    ra = rem_ref[...]
    row_ids = lax.broadcasted_iota(jnp.int32, (1, N, 1), 1)
    mask = ra >= row_ids
    neg = jnp.where(mask, rows, -jnp.inf)
    mx = jnp.max(neg, axis=2, keepdims=True)
    e = jnp.where(mask, jnp.exp(rows - mx), 0.0)
    o_ref[...] = e / jnp.sum(e, axis=2, keepdims=True)


def _phase2(s, removed):
    rem3 = removed.reshape(B, 1, M)
    return pl.pallas_call(
        _p2_kernel,
        grid=(B // TC_BATCH,),
        in_specs=[
            pl.BlockSpec((TC_BATCH, N, M), lambda bb: (bb, 0, 0)),
            pl.BlockSpec((TC_BATCH, 1, M), lambda bb: (bb, 0, 0)),
        ],
        out_specs=pl.BlockSpec((TC_BATCH, N, M), lambda bb: (bb, 0, 0)),
        out_shape=jax.ShapeDtypeStruct((B, N, M), jnp.float32),
        name="p2_softmax",
    )(s, rem3)


def kernel(s):
    removed = _phase1(s)
    return _phase2(s, removed)


# speedup vs baseline: 1.7821x; 1.0972x over previous
import functools

import jax
import jax.numpy as jnp
from jax import lax
from jax.experimental import pallas as pl
from jax.experimental.pallas import tpu as pltpu
from jax.experimental.pallas import tpu_sc as plsc

B, N, M = 32, 512, 512
L = 16
NC, NS = 2, 16
ROWS_BLK = 64
TC_BATCH = 2
BIG = 2**30


def _p1_body(s_hbm, removed_hbm, buf0, buf1, pen, rem, sem0, sem1):
    b = lax.axis_index("s") * NC + lax.axis_index("c")
    lane_iota = lax.broadcasted_iota(jnp.int32, (L,), 0)

    for k in range(M // L):
        pen[pl.ds(k * L, L)] = jnp.zeros((L,), jnp.float32)
        rem[pl.ds(k * L, L)] = jnp.zeros((L,), jnp.int32)

    bufs = (buf0, buf1)
    sems = (sem0, sem1)
    n_blk = N // ROWS_BLK
    copies = [None] * n_blk
    copies[0] = pltpu.async_copy(
        s_hbm.at[b, pl.ds(0, ROWS_BLK)], bufs[0], sems[0])

    n_grp = 4
    per_grp = M // L // n_grp

    for blk in range(n_blk):
        rowbuf = bufs[blk % 2]
        copies[blk].wait()
        if blk + 1 < n_blk:
            copies[blk + 1] = pltpu.async_copy(
                s_hbm.at[b, pl.ds((blk + 1) * ROWS_BLK, ROWS_BLK)],
                bufs[(blk + 1) % 2], sems[(blk + 1) % 2])

        def row_body(r, carry, rowbuf=rowbuf, blk=blk):
            i = blk * ROWS_BLK + r
            accs = []
            for g in range(n_grp):
                bv = jnp.full((L,), -jnp.inf, jnp.float32)
                bi = jnp.zeros((L,), jnp.int32)
                for k in range(per_grp):
                    kk = g * per_grp + k
                    v = rowbuf[r, pl.ds(kk * L, L)] + pen[pl.ds(kk * L, L)]
                    gt = v > bv
                    bv = jnp.where(gt, v, bv)
                    bi = jnp.where(gt, lane_iota + (kk * L), bi)
                accs.append((bv, bi))
            while len(accs) > 1:
                nxt = []
                for (av, ai), (bv, bi) in zip(accs[::2], accs[1::2]):
                    gt = bv > av
                    nxt.append((jnp.where(gt, bv, av), jnp.where(gt, bi, ai)))
                accs = nxt
            best_v, best_i = accs[0]
            mx = jnp.max(best_v)
            cand = jnp.where(best_v == mx, best_i, jnp.int32(BIG))
            idx = jnp.min(cand)
            idxv = jnp.full((L,), idx, jnp.int32)
            lane0 = lane_iota == 0
            plsc.store_scatter(
                pen, [idxv], jnp.full((L,), -jnp.inf, jnp.float32), mask=lane0)
            plsc.store_scatter(
                rem, [idxv], jnp.full((L,), i, jnp.int32), mask=lane0)
            return carry

        lax.fori_loop(0, ROWS_BLK, row_body, 0)

    pltpu.sync_copy(rem, removed_hbm.at[b])


def _phase1(s):
    mesh = plsc.VectorSubcoreMesh(core_axis_name="c", subcore_axis_name="s")
    kern = functools.partial(
        pl.kernel,
        mesh=mesh,
        out_type=jax.ShapeDtypeStruct((B, M), jnp.int32),
        scratch_types=[
            pltpu.VMEM((ROWS_BLK, M), jnp.float32),
            pltpu.VMEM((ROWS_BLK, M), jnp.float32),
            pltpu.VMEM((M,), jnp.float32),
            pltpu.VMEM((M,), jnp.int32),
            pltpu.SemaphoreType.DMA,
            pltpu.SemaphoreType.DMA,
        ],
        compiler_params=pltpu.CompilerParams(needs_layout_passes=False),
        name="p1_greedy",
    )(_p1_body)
    return kern(s)


def _p2_kernel(s_ref, rem_ref, o_ref):
    rows = s_ref[...]
    ra = rem_ref[...]
    row_ids = lax.broadcasted_iota(jnp.int32, (1, N, 1), 1)
    mask = ra >= row_ids
    neg = jnp.where(mask, rows, -jnp.inf)
    mx = jnp.max(neg, axis=2, keepdims=True)
    e = jnp.where(mask, jnp.exp(rows - mx), 0.0)
    o_ref[...] = e / jnp.sum(e, axis=2, keepdims=True)


def _phase2(s, removed):
    rem3 = removed.reshape(B, 1, M)
    return pl.pallas_call(
        _p2_kernel,
        grid=(B // TC_BATCH,),
        in_specs=[
            pl.BlockSpec((TC_BATCH, N, M), lambda bb: (bb, 0, 0)),
            pl.BlockSpec((TC_BATCH, 1, M), lambda bb: (bb, 0, 0)),
        ],
        out_specs=pl.BlockSpec((TC_BATCH, N, M), lambda bb: (bb, 0, 0)),
        out_shape=jax.ShapeDtypeStruct((B, N, M), jnp.float32),
        name="p2_softmax",
    )(s, rem3)


def kernel(s):
    removed = _phase1(s)
    return _phase2(s, removed)


# speedup vs baseline: 1.8534x; 1.0400x over previous
import functools

import jax
import jax.numpy as jnp
from jax import lax
from jax.experimental import pallas as pl
from jax.experimental.pallas import tpu as pltpu
from jax.experimental.pallas import tpu_sc as plsc

B, N, M = 32, 512, 512
L = 16
NC, NS = 2, 16
ROWS_BLK = 64
TC_BATCH = 4
BIG = 2**30


def _p1_body(s_hbm, removed_hbm, buf0, buf1, pen, rem, sem0, sem1):
    b = lax.axis_index("s") * NC + lax.axis_index("c")
    lane_iota = lax.broadcasted_iota(jnp.int32, (L,), 0)

    for k in range(M // L):
        pen[pl.ds(k * L, L)] = jnp.zeros((L,), jnp.float32)
        rem[pl.ds(k * L, L)] = jnp.zeros((L,), jnp.int32)

    bufs = (buf0, buf1)
    sems = (sem0, sem1)
    n_blk = N // ROWS_BLK
    copies = [None] * n_blk
    copies[0] = pltpu.async_copy(
        s_hbm.at[b, pl.ds(0, ROWS_BLK)], bufs[0], sems[0])

    n_grp = 4
    per_grp = M // L // n_grp

    for blk in range(n_blk):
        rowbuf = bufs[blk % 2]
        copies[blk].wait()
        if blk + 1 < n_blk:
            copies[blk + 1] = pltpu.async_copy(
                s_hbm.at[b, pl.ds((blk + 1) * ROWS_BLK, ROWS_BLK)],
                bufs[(blk + 1) % 2], sems[(blk + 1) % 2])

        def row_body(r, carry, rowbuf=rowbuf, blk=blk):
            i = blk * ROWS_BLK + r
            accs = []
            for g in range(n_grp):
                bv = jnp.full((L,), -jnp.inf, jnp.float32)
                bi = jnp.zeros((L,), jnp.int32)
                for k in range(per_grp):
                    kk = g * per_grp + k
                    v = rowbuf[r, pl.ds(kk * L, L)] + pen[pl.ds(kk * L, L)]
                    gt = v > bv
                    bv = jnp.where(gt, v, bv)
                    bi = jnp.where(gt, lane_iota + (kk * L), bi)
                accs.append((bv, bi))
            while len(accs) > 1:
                nxt = []
                for (av, ai), (bv, bi) in zip(accs[::2], accs[1::2]):
                    gt = bv > av
                    nxt.append((jnp.where(gt, bv, av), jnp.where(gt, bi, ai)))
                accs = nxt
            best_v, best_i = accs[0]
            mx = jnp.max(best_v)
            cand = jnp.where(best_v == mx, best_i, jnp.int32(BIG))
            idx = jnp.min(cand)
            idxv = jnp.full((L,), idx, jnp.int32)
            lane0 = lane_iota == 0
            plsc.store_scatter(
                pen, [idxv], jnp.full((L,), -jnp.inf, jnp.float32), mask=lane0)
            plsc.store_scatter(
                rem, [idxv], jnp.full((L,), i, jnp.int32), mask=lane0)
            return carry

        lax.fori_loop(0, ROWS_BLK, row_body, 0)

    pltpu.sync_copy(rem, removed_hbm.at[b])


def _phase1(s):
    mesh = plsc.VectorSubcoreMesh(core_axis_name="c", subcore_axis_name="s")
    kern = functools.partial(
        pl.kernel,
        mesh=mesh,
        out_type=jax.ShapeDtypeStruct((B, M), jnp.int32),
        scratch_types=[
            pltpu.VMEM((ROWS_BLK, M), jnp.float32),
            pltpu.VMEM((ROWS_BLK, M), jnp.float32),
            pltpu.VMEM((M,), jnp.float32),
            pltpu.VMEM((M,), jnp.int32),
            pltpu.SemaphoreType.DMA,
            pltpu.SemaphoreType.DMA,
        ],
        compiler_params=pltpu.CompilerParams(needs_layout_passes=False),
        name="p1_greedy",
    )(_p1_body)
    return kern(s)


def _p2_kernel(s_ref, rem_ref, o_ref):
    rows = s_ref[...]
    ra = rem_ref[...]
    row_ids = lax.broadcasted_iota(jnp.int32, (1, N, 1), 1)
    mask = ra >= row_ids
    neg = jnp.where(mask, rows, -jnp.inf)
    mx = jnp.max(neg, axis=2, keepdims=True)
    e = jnp.where(mask, jnp.exp(rows - mx), 0.0)
    o_ref[...] = e / jnp.sum(e, axis=2, keepdims=True)


def _phase2(s, removed):
    rem3 = removed.reshape(B, 1, M)
    return pl.pallas_call(
        _p2_kernel,
        grid=(B // TC_BATCH,),
        in_specs=[
            pl.BlockSpec((TC_BATCH, N, M), lambda bb: (bb, 0, 0)),
            pl.BlockSpec((TC_BATCH, 1, M), lambda bb: (bb, 0, 0)),
        ],
        out_specs=pl.BlockSpec((TC_BATCH, N, M), lambda bb: (bb, 0, 0)),
        out_shape=jax.ShapeDtypeStruct((B, N, M), jnp.float32),
        name="p2_softmax",
    )(s, rem3)


def kernel(s):
    removed = _phase1(s)
    return _phase2(s, removed)
